# baseline (speedup 1.0000x reference)
import functools

import jax
import jax.numpy as jnp
from jax import lax
from jax.experimental import pallas as pl
from jax.experimental.pallas import tpu as pltpu
from jax.experimental.pallas import tpu_sc as plsc

N = 1024
F = 64
K = 16
L = 16
NC = 2
NS = 16
NW = NC * NS
JW = N // NW
FU = 8
NCHUNK = N // L


def _sc_body(xT_hbm, adjT_hbm, x_hbm, out_hbm, xT_v, row_v, buf_v,
             sem_row, sem_out):
    wid = lax.axis_index("s") * NC + lax.axis_index("c")
    j0 = wid * JW
    pltpu.sync_copy(xT_hbm, xT_v)
    pltpu.sync_copy(adjT_hbm.at[j0], row_v.at[0])

    rows_idx = lax.iota(jnp.int32, L) + 1
    neg_inf = jnp.full((L,), -jnp.inf, jnp.float32)

    def j_body(jj, carry):
        j = j0 + jj
        slot = jj % 2
        nslot = (jj + 1) % 2
        j_next = jnp.minimum(j + 1, j0 + JW - 1)
        pref = pltpu.async_copy(adjT_hbm.at[j_next], row_v.at[nslot], sem_row)

        @pl.when(jj >= 1)
        def _():
            pltpu.make_async_copy(
                buf_v.at[nslot], out_hbm.at[jnp.maximum(j - 1, j0)], sem_out
            ).wait()

        pltpu.sync_copy(x_hbm.at[j], buf_v.at[slot, 0])

        for fg in range(F // FU):
            def chunk_body(c, runs):
                base = c * L
                a = row_v[slot, pl.ds(base, L)]
                new_runs = []
                for u in range(FU):
                    xv = xT_v[fg * FU + u, pl.ds(base, L)]
                    p, _ = plsc.sort_key_val(a * xv, a * xv)
                    m = jnp.maximum(runs[u], p)
                    r, _ = plsc.sort_key_val(m, m, descending=True)
                    new_runs.append(r)
                return tuple(new_runs)

            runs = plsc.parallel_loop(
                0, NCHUNK, 1, unroll=8,
                carry=tuple(neg_inf for _ in range(FU)),
            )(chunk_body)
            for u in range(FU):
                cols = jnp.full((L,), fg * FU + u, jnp.int32)
                plsc.store_scatter(buf_v.at[slot], [rows_idx, cols], runs[u])

        pltpu.async_copy(buf_v.at[slot], out_hbm.at[j], sem_out)
        pref.wait()
        return carry

    lax.fori_loop(0, JW, j_body, 0)
    pltpu.make_async_copy(
        buf_v.at[(JW - 1) % 2], out_hbm.at[j0 + JW - 1], sem_out
    ).wait()


def kernel(x, adj):
    xT = jnp.transpose(x)
    adjT = jnp.transpose(adj)

    mesh = plsc.VectorSubcoreMesh(core_axis_name="c", subcore_axis_name="s")
    run = pl.kernel(
        _sc_body,
        out_type=jax.ShapeDtypeStruct((N, K + 1, F), jnp.float32),
        mesh=mesh,
        compiler_params=pltpu.CompilerParams(needs_layout_passes=False),
        scratch_types=[
            pltpu.VMEM((F, N), jnp.float32),
            pltpu.VMEM((2, N), jnp.float32),
            pltpu.VMEM((2, K + 1, F), jnp.float32),
            pltpu.SemaphoreType.DMA,
            pltpu.SemaphoreType.DMA,
        ],
    )
    return run(xT, adjT, x)


# speedup vs baseline: 1.1787x; 1.1787x over previous
import functools

import jax
import jax.numpy as jnp
from jax import lax
from jax.experimental import pallas as pl
from jax.experimental.pallas import tpu as pltpu
from jax.experimental.pallas import tpu_sc as plsc

N = 1024
F = 64
K = 16
L = 16
NC = 2
NS = 16
NW = NC * NS
JW = N // NW
FU = 8
NCHUNK = N // L


def _sc_body(xT_hbm, adjT_hbm, x_hbm, out_hbm, xT_v, row_v, buf_v,
             sem_row, sem_out):
    wid = lax.axis_index("s") * NC + lax.axis_index("c")
    j0 = wid * JW
    pltpu.sync_copy(xT_hbm, xT_v)
    pltpu.sync_copy(adjT_hbm.at[j0], row_v.at[0])

    rows_idx = lax.iota(jnp.int32, L) + 1
    neg_inf = jnp.full((L,), -jnp.inf, jnp.float32)

    def one_node(jj, slot):
        nslot = 1 - slot
        j = j0 + jj
        j_next = jnp.minimum(j + 1, j0 + JW - 1)
        pref = pltpu.async_copy(adjT_hbm.at[j_next], row_v.at[nslot], sem_row)

        @pl.when(jj >= 1)
        def _():
            pltpu.make_async_copy(
                buf_v.at[nslot], out_hbm.at[jnp.maximum(j - 1, j0)], sem_out
            ).wait()

        xrow = pltpu.async_copy(x_hbm.at[j], buf_v.at[slot, 0], sem_row)

        for fg in range(F // FU):
            def chunk_body(c, runs):
                base = c * L
                a = row_v[slot, pl.ds(base, L)]
                new_runs = []
                for u in range(FU):
                    xv = xT_v[fg * FU + u, pl.ds(base, L)]
                    p, _ = plsc.sort_key_val(a * xv, a * xv)
                    m = jnp.maximum(runs[u], p)
                    r, _ = plsc.sort_key_val(m, m, descending=True)
                    new_runs.append(r)
                return tuple(new_runs)

            runs = plsc.parallel_loop(
                0, NCHUNK, 1, unroll=4,
                carry=tuple(neg_inf for _ in range(FU)),
            )(chunk_body)
            for u in range(FU):
                cols = jnp.full((L,), fg * FU + u, jnp.int32)
                plsc.store_scatter(buf_v.at[slot], [rows_idx, cols], runs[u])

        xrow.wait()
        pltpu.async_copy(buf_v.at[slot], out_hbm.at[j], sem_out)
        pref.wait()

    def j_body(jp, carry):
        one_node(2 * jp, 0)
        one_node(2 * jp + 1, 1)
        return carry

    lax.fori_loop(0, JW // 2, j_body, 0)
    pltpu.make_async_copy(
        buf_v.at[(JW - 1) % 2], out_hbm.at[j0 + JW - 1], sem_out
    ).wait()


def kernel(x, adj):
    xT = jnp.transpose(x)
    adjT = jnp.transpose(adj)

    mesh = plsc.VectorSubcoreMesh(core_axis_name="c", subcore_axis_name="s")
    run = pl.kernel(
        _sc_body,
        out_type=jax.ShapeDtypeStruct((N, K + 1, F), jnp.float32),
        mesh=mesh,
        compiler_params=pltpu.CompilerParams(needs_layout_passes=False),
        scratch_types=[
            pltpu.VMEM((F, N), jnp.float32),
            pltpu.VMEM((2, N), jnp.float32),
            pltpu.VMEM((2, K + 1, F), jnp.float32),
            pltpu.SemaphoreType.DMA,
            pltpu.SemaphoreType.DMA,
        ],
    )
    return run(xT, adjT, x)


# speedup vs baseline: 1.3634x; 1.1566x over previous
import jax
import jax.numpy as jnp
from jax import lax
from jax.experimental import pallas as pl
from jax.experimental.pallas import tpu as pltpu
from jax.experimental.pallas import tpu_sc as plsc

N = 1024
F = 64
K = 16
L = 16
NC = 2
NS = 16
NW = NC * NS
JSC = 832
JW = JSC // NW
JT = N - JSC
BJ = 8
FU = 8
NCHUNK = N // L


def _sc_body(xT_hbm, adjT_hbm, x_hbm, out_hbm, xT_v, row_v, buf_v,
             sem_row, sem_out):
    wid = lax.axis_index("s") * NC + lax.axis_index("c")
    j0 = wid * JW
    pltpu.sync_copy(xT_hbm, xT_v)
    pltpu.sync_copy(adjT_hbm.at[j0], row_v.at[0])

    rows_idx = lax.iota(jnp.int32, L) + 1
    neg_inf = jnp.full((L,), -jnp.inf, jnp.float32)

    def one_node(jj, slot):
        nslot = 1 - slot
        j = j0 + jj
        j_next = jnp.minimum(j + 1, j0 + JW - 1)
        pref = pltpu.async_copy(adjT_hbm.at[j_next], row_v.at[nslot], sem_row)

        @pl.when(jj >= 1)
        def _():
            pltpu.make_async_copy(
                buf_v.at[nslot], out_hbm.at[jnp.maximum(j - 1, j0)], sem_out
            ).wait()

        xrow = pltpu.async_copy(x_hbm.at[j], buf_v.at[slot, 0], sem_row)

        for fg in range(F // FU):
            def chunk_body(c, runs):
                base = c * L
                a = row_v[slot, pl.ds(base, L)]
                new_runs = []
                for u in range(FU):
                    xv = xT_v[fg * FU + u, pl.ds(base, L)]
                    p, _ = plsc.sort_key_val(a * xv, a * xv)
                    m = jnp.maximum(runs[u], p)
                    r, _ = plsc.sort_key_val(m, m, descending=True)
                    new_runs.append(r)
                return tuple(new_runs)

            runs = plsc.parallel_loop(
                0, NCHUNK, 1, unroll=4,
                carry=tuple(neg_inf for _ in range(FU)),
            )(chunk_body)
            for u in range(FU):
                cols = jnp.full((L,), fg * FU + u, jnp.int32)
                plsc.store_scatter(buf_v.at[slot], [rows_idx, cols], runs[u])

        xrow.wait()
        pltpu.async_copy(buf_v.at[slot], out_hbm.at[j], sem_out)
        pref.wait()

    def j_body(jp, carry):
        one_node(2 * jp, 0)
        one_node(2 * jp + 1, 1)
        return carry

    lax.fori_loop(0, JW // 2, j_body, 0)
    pltpu.make_async_copy(
        buf_v.at[(JW - 1) % 2], out_hbm.at[j0 + JW - 1], sem_out
    ).wait()


def _tc_body(adjb_ref, xT_ref, xb_ref, out_ref):
    out_ref[:, 0, :] = xb_ref[...]
    lane = lax.broadcasted_iota(jnp.int32, (F, N), 1) & 1023
    for jb in range(BJ):
        row = adjb_ref[jb, :]
        p = xT_ref[...] * row[None, :]
        b = lax.bitcast_convert_type(p, jnp.int32)
        b = (b & jnp.int32(~1023)) | lane
        u = jnp.where(b >= 0, b, b ^ jnp.int32(0x7FFFFFFF))
        tops = []
        for t in range(K):
            m = jnp.max(u, axis=1, keepdims=True)
            u = jnp.where(u == m, jnp.int32(-0x80000000), u)
            tops.append(m)
        mu = jnp.concatenate(tops, axis=1)
        mb = jnp.where(mu >= 0, mu, mu ^ jnp.int32(0x7FFFFFFF))
        vals = lax.bitcast_convert_type(mb, jnp.float32)
        out_ref[jb, 1:, :] = vals.T


def kernel(x, adj):
    xT = jnp.transpose(x)
    adjT = jnp.transpose(adj)

    mesh = plsc.VectorSubcoreMesh(core_axis_name="c", subcore_axis_name="s")
    sc_run = pl.kernel(
        _sc_body,
        out_type=jax.ShapeDtypeStruct((JSC, K + 1, F), jnp.float32),
        mesh=mesh,
        compiler_params=pltpu.CompilerParams(needs_layout_passes=False),
        scratch_types=[
            pltpu.VMEM((F, N), jnp.float32),
            pltpu.VMEM((2, N), jnp.float32),
            pltpu.VMEM((2, K + 1, F), jnp.float32),
            pltpu.SemaphoreType.DMA,
            pltpu.SemaphoreType.DMA,
        ],
    )

    tc_run = pl.pallas_call(
        _tc_body,
        grid=(JT // BJ,),
        in_specs=[
            pl.BlockSpec((BJ, N), lambda g: (g, 0)),
            pl.BlockSpec((F, N), lambda g: (0, 0)),
            pl.BlockSpec((BJ, F), lambda g: (g, 0)),
        ],
        out_specs=pl.BlockSpec((BJ, K + 1, F), lambda g: (g, 0, 0)),
        out_shape=jax.ShapeDtypeStruct((JT, K + 1, F), jnp.float32),
    )

    out_sc = sc_run(xT, adjT, x)
    out_tc = tc_run(adjT[JSC:], xT, x[JSC:])
    return jnp.concatenate([out_sc, out_tc], axis=0)
